# Initial kernel scaffold; baseline (speedup 1.0000x reference)
import jax
import jax.numpy as jnp
from jax.experimental import pallas as pl


def kernel(hidden_states, router_w, gate_up_proj, down_proj, shared_gate, shared_up, shared_down):
    raise NotImplementedError("write your pallas kernel here")


# speedup vs baseline: 1.0504x; 1.0504x over previous
import functools

import jax
import jax.numpy as jnp
from jax.experimental import pallas as pl

def _silu(x):
    return x * jax.nn.sigmoid(x)


def _dot(a, b):
    return jnp.dot(a, b, preferred_element_type=jnp.float32)


def _router_body(hs_ref, rwt_ref, scores_et_ref, scores_te_ref):
    E = rwt_ref.shape[1]
    T = hs_ref.shape[0]
    logits = _dot(hs_ref[...], rwt_ref[...])
    ie = jax.lax.broadcasted_iota(jnp.int32, (T, E), 1)
    m1 = jnp.max(logits, axis=1, keepdims=True)
    i1 = jnp.min(jnp.where(logits == m1, ie, E), axis=1, keepdims=True)
    mask1 = ie == i1
    l2 = jnp.where(mask1, -jnp.inf, logits)
    m2 = jnp.max(l2, axis=1, keepdims=True)
    i2 = jnp.min(jnp.where(l2 == m2, ie, E), axis=1, keepdims=True)
    sel = mask1 | (ie == i2)
    scores = jnp.where(sel, jax.nn.sigmoid(logits), 0.0)
    scores_te_ref[...] = scores
    scores_et_ref[...] = scores.T


def _shared_body(hs_ref, sg_ref, su_ref, sd_ref, out_ref, *, bt, n_f):
    f = pl.program_id(0)
    tb = pl.program_id(1)
    x = hs_ref[...]
    g = _dot(x, sg_ref[...])
    u = _dot(x, su_ref[...])
    y = _dot(_silu(g) * u, sd_ref[...])
    rows = pl.ds(tb * bt, bt)

    @pl.when(f == 0)
    def _init():
        out_ref[rows, :] = y

    @pl.when(f != 0)
    def _acc():
        out_ref[rows, :] += y


def _expert_body(hs_ref, sc_ref, gw_ref, uw_ref, dn_ref, sh_ref, out_ref,
                 *, bt):
    e = pl.program_id(0)
    f = pl.program_id(1)
    tb = pl.program_id(2)
    x = hs_ref[...]
    sc = sc_ref[...]
    lane = jax.lax.broadcasted_iota(jnp.int32, sc.shape, 1)
    s = jnp.sum(jnp.where(lane == e, sc, 0.0), axis=1, keepdims=True)
    xs = x * s
    g = _dot(xs, gw_ref[0])
    u = _dot(xs, uw_ref[0])
    y = _dot(u * _silu(g), dn_ref[0])
    rows = pl.ds(tb * bt, bt)
    first = jnp.logical_and(e == 0, f == 0)

    @pl.when(first)
    def _init():
        out_ref[rows, :] = sh_ref[rows, :] + y

    @pl.when(jnp.logical_not(first))
    def _acc():
        out_ref[rows, :] += y


def kernel(hidden_states, router_w, gate_up_proj, down_proj, shared_gate,
           shared_up, shared_down):
    d = hidden_states.shape[-1]
    hs = hidden_states.reshape(-1, d)
    T = hs.shape[0]
    E = router_w.shape[0]
    F = down_proj.shape[1]

    scores_et, scores_te = pl.pallas_call(
        _router_body,
        out_shape=(
            jax.ShapeDtypeStruct((E, T), jnp.float32),
            jax.ShapeDtypeStruct((T, E), jnp.float32),
        ),
    )(hs, router_w.T)

    BT = 512
    FB = 512
    n_tb = T // BT
    n_f = F // FB

    shared_out = pl.pallas_call(
        functools.partial(_shared_body, bt=BT, n_f=n_f),
        grid=(n_f, n_tb),
        in_specs=[
            pl.BlockSpec((BT, d), lambda f, tb: (tb, 0)),
            pl.BlockSpec((d, FB), lambda f, tb: (0, f)),
            pl.BlockSpec((d, FB), lambda f, tb: (0, f)),
            pl.BlockSpec((FB, d), lambda f, tb: (f, 0)),
        ],
        out_specs=pl.BlockSpec((T, d), lambda f, tb: (0, 0)),
        out_shape=jax.ShapeDtypeStruct((T, d), jnp.float32),
    )(hs, shared_gate, shared_up, shared_down)

    out = pl.pallas_call(
        functools.partial(_expert_body, bt=BT),
        grid=(E, n_f, n_tb),
        in_specs=[
            pl.BlockSpec((BT, d), lambda e, f, tb: (tb, 0)),
            pl.BlockSpec((BT, E), lambda e, f, tb: (tb, 0)),
            pl.BlockSpec((1, d, FB), lambda e, f, tb: (e, 0, f)),
            pl.BlockSpec((1, d, FB), lambda e, f, tb: (e, 0, n_f + f)),
            pl.BlockSpec((1, FB, d), lambda e, f, tb: (e, f, 0)),
            pl.BlockSpec((T, d), lambda e, f, tb: (0, 0)),
        ],
        out_specs=pl.BlockSpec((T, d), lambda e, f, tb: (0, 0)),
        out_shape=jax.ShapeDtypeStruct((T, d), jnp.float32),
    )(hs, scores_te, gate_up_proj, gate_up_proj, down_proj, shared_out)

    return out, scores_et


# speedup vs baseline: 1.4201x; 1.3519x over previous
import functools

import jax
import jax.numpy as jnp
from jax.experimental import pallas as pl
from jax.experimental.pallas import tpu as pltpu


def _silu(x):
    return x * jax.nn.sigmoid(x)


def _dot(a, b):
    return jnp.dot(a, b, preferred_element_type=jnp.float32)


def _router_body(hs_ref, rwt_ref, scores_et_ref, slots_ref, scales_ref,
                 ge_ref, nu_ref, *, bt, g_max):
    T = hs_ref.shape[0]
    E = rwt_ref.shape[1]
    f32 = jnp.float32
    logits = _dot(hs_ref[...], rwt_ref[...])
    ie = jax.lax.broadcasted_iota(jnp.int32, (T, E), 1)
    m1 = jnp.max(logits, axis=1, keepdims=True)
    i1 = jnp.min(jnp.where(logits == m1, ie, E), axis=1, keepdims=True)
    mask1 = ie == i1
    l2 = jnp.where(mask1, -jnp.inf, logits)
    m2 = jnp.max(l2, axis=1, keepdims=True)
    i2 = jnp.min(jnp.where(l2 == m2, ie, E), axis=1, keepdims=True)
    mask2 = ie == i2
    sel = mask1 | mask2
    scores = jnp.where(sel, jax.nn.sigmoid(logits), 0.0)
    scores_et_ref[...] = scores.T

    oh1 = mask1.astype(f32)
    oh2 = mask2.astype(f32)
    self32 = sel.astype(f32)

    cb = 512
    nb = T // cb
    rr = jax.lax.broadcasted_iota(jnp.int32, (cb, cb), 0)
    cc = jax.lax.broadcasted_iota(jnp.int32, (cb, cb), 1)
    tril_s = (cc < rr).astype(f32)
    carry = jnp.zeros((1, E), f32)
    blocks = []
    for b in range(nb):
        blk = self32[b * cb:(b + 1) * cb, :]
        blocks.append(_dot(tril_s, blk) + carry)
        carry = carry + jnp.sum(blk, axis=0, keepdims=True)
    count_c = jnp.concatenate(blocks, axis=0)
    counts_row = carry

    ng_row = jnp.floor((counts_row + (bt - 1)) * (1.0 / bt))
    er = jax.lax.broadcasted_iota(jnp.int32, (E, E), 0)
    ec = jax.lax.broadcasted_iota(jnp.int32, (E, E), 1)
    triu_s = (er < ec).astype(f32)
    gstart_row = _dot(ng_row, triu_s)
    off_row = gstart_row * bt

    slot_base = count_c + off_row
    slot1 = jnp.sum(oh1 * slot_base, axis=1, keepdims=True)
    slot2 = jnp.sum(oh2 * slot_base, axis=1, keepdims=True)
    scale1 = jnp.sum(oh1 * scores, axis=1, keepdims=True)
    scale2 = jnp.sum(oh2 * scores, axis=1, keepdims=True)
    slots_ref[...] = jnp.concatenate([slot1, slot2], axis=1)
    scales_ref[...] = jnp.concatenate([scale1, scale2], axis=1)

    ones_t = jnp.ones((T, 1), f32)
    counts_col = jax.lax.dot_general(
        self32, ones_t, (((0,), (0,)), ((), ())),
        preferred_element_type=f32)
    ng_col = jnp.floor((counts_col + (bt - 1)) * (1.0 / bt))
    tril_se = (ec < er).astype(f32)
    gstart_col = _dot(tril_se, ng_col)
    gg = jax.lax.broadcasted_iota(jnp.int32, (E, g_max), 1).astype(f32)
    m = (gstart_col <= gg).astype(f32)
    ones_e = jnp.ones((1, E), f32)
    ge_row = _dot(ones_e, m) - 1.0
    n_used = _dot(ones_e, ng_col)
    ge_ref[...] = ge_row.astype(jnp.int32)
    nu_ref[...] = n_used.astype(jnp.int32)


def _shared_body(hs_ref, sg_ref, su_ref, sd_ref, out_ref, *, bt):
    f = pl.program_id(0)
    tb = pl.program_id(1)
    x = hs_ref[...]
    g = _dot(x, sg_ref[...])
    u = _dot(x, su_ref[...])
    y = _dot(_silu(g) * u, sd_ref[...])
    rows = pl.ds(tb * bt, bt)

    @pl.when(f == 0)
    def _init():
        out_ref[rows, :] = y

    @pl.when(f != 0)
    def _acc():
        out_ref[rows, :] += y


def _group_body(ge_ref, nu_ref, hs_ref, slots_ref, scales_ref, gw_ref,
                uw_ref, dn_ref, sh_ref, out_ref, x_s, y_s, *, bt, n_f):
    g = pl.program_id(0)
    f = pl.program_id(1)

    @pl.when(jnp.logical_and(g == 0, f == 0))
    def _init():
        out_ref[...] = sh_ref[...]

    @pl.when(g < nu_ref[0])
    def _work():
        T = slots_ref.shape[0]
        base = (g * bt).astype(jnp.float32)
        lane_b = jax.lax.broadcasted_iota(
            jnp.int32, (T, bt), 1).astype(jnp.float32) + base
        s1 = slots_ref[:, 0:1]
        s2 = slots_ref[:, 1:2]
        m1 = s1 == lane_b
        m2 = s2 == lane_b

        @pl.when(f == 0)
        def _gather():
            pt = (jnp.where(m1, scales_ref[:, 0:1], 0.0) +
                  jnp.where(m2, scales_ref[:, 1:2], 0.0))
            x_s[...] = jax.lax.dot_general(
                pt, hs_ref[...], (((0,), (0,)), ((), ())),
                preferred_element_type=jnp.float32)

        x = x_s[...]
        gb = _dot(x, gw_ref[0])
        ub = _dot(x, uw_ref[0])
        yp = _dot(ub * _silu(gb), dn_ref[0])

        @pl.when(f == 0)
        def _y0():
            y_s[...] = yp

        @pl.when(f != 0)
        def _yacc():
            y_s[...] += yp

        @pl.when(f == n_f - 1)
        def _scatter():
            q = m1.astype(jnp.float32) + m2.astype(jnp.float32)
            out_ref[...] += _dot(q, y_s[...])


def kernel(hidden_states, router_w, gate_up_proj, down_proj, shared_gate,
           shared_up, shared_down):
    d = hidden_states.shape[-1]
    hs = hidden_states.reshape(-1, d)
    T = hs.shape[0]
    E = router_w.shape[0]
    F = down_proj.shape[1]

    BT = 512
    G = 16
    FB = 512
    n_f = F // FB

    scores_et, slots, scales, ge2d, nu2d = pl.pallas_call(
        functools.partial(_router_body, bt=BT, g_max=G),
        out_shape=(
            jax.ShapeDtypeStruct((E, T), jnp.float32),
            jax.ShapeDtypeStruct((T, 2), jnp.float32),
            jax.ShapeDtypeStruct((T, 2), jnp.float32),
            jax.ShapeDtypeStruct((1, G), jnp.int32),
            jax.ShapeDtypeStruct((1, 1), jnp.int32),
        ),
    )(hs, router_w.T)
    ge = ge2d.reshape(G)
    nu = nu2d.reshape(1)

    shared_out = pl.pallas_call(
        functools.partial(_shared_body, bt=BT),
        grid=(n_f, T // BT),
        in_specs=[
            pl.BlockSpec((BT, d), lambda f, tb: (tb, 0)),
            pl.BlockSpec((d, FB), lambda f, tb: (0, f)),
            pl.BlockSpec((d, FB), lambda f, tb: (0, f)),
            pl.BlockSpec((FB, d), lambda f, tb: (f, 0)),
        ],
        out_specs=pl.BlockSpec((T, d), lambda f, tb: (0, 0)),
        out_shape=jax.ShapeDtypeStruct((T, d), jnp.float32),
    )(hs, shared_gate, shared_up, shared_down)

    grid_spec = pltpu.PrefetchScalarGridSpec(
        num_scalar_prefetch=2,
        grid=(G, n_f),
        in_specs=[
            pl.BlockSpec((T, d), lambda g, f, ge, nu: (0, 0)),
            pl.BlockSpec((T, 2), lambda g, f, ge, nu: (0, 0)),
            pl.BlockSpec((T, 2), lambda g, f, ge, nu: (0, 0)),
            pl.BlockSpec((1, d, FB), lambda g, f, ge, nu: (ge[g], 0, f)),
            pl.BlockSpec((1, d, FB),
                         lambda g, f, ge, nu: (ge[g], 0, n_f + f)),
            pl.BlockSpec((1, FB, d), lambda g, f, ge, nu: (ge[g], f, 0)),
            pl.BlockSpec((T, d), lambda g, f, ge, nu: (0, 0)),
        ],
        out_specs=pl.BlockSpec((T, d), lambda g, f, ge, nu: (0, 0)),
        scratch_shapes=[
            pltpu.VMEM((BT, d), jnp.float32),
            pltpu.VMEM((BT, d), jnp.float32),
        ],
    )
    out = pl.pallas_call(
        functools.partial(_group_body, bt=BT, n_f=n_f),
        grid_spec=grid_spec,
        out_shape=jax.ShapeDtypeStruct((T, d), jnp.float32),
    )(ge, nu, hs, slots, scales, gate_up_proj, gate_up_proj, down_proj,
      shared_out)

    return out, scores_et


# speedup vs baseline: 1.4588x; 1.0273x over previous
import functools

import jax
import jax.numpy as jnp
from jax import lax
from jax.experimental import pallas as pl
from jax.experimental.pallas import tpu as pltpu
from jax.experimental.pallas import tpu_sc as plsc

_NC = 2
_NS = 16
_NW = _NC * _NS
_L = 16


def _silu(x):
    return x * jax.nn.sigmoid(x)


def _dot(a, b):
    return jnp.dot(a, b, preferred_element_type=jnp.float32)


def _router_body(hs_ref, rwt_ref, scores_et_ref, h2a_ref, h2b_ref,
                 slots_ref, ge_ref, nu_ref, *, bt, g_max):
    T = hs_ref.shape[0]
    E = rwt_ref.shape[1]
    f32 = jnp.float32
    hs = hs_ref[...]
    logits = _dot(hs, rwt_ref[...])
    ie = jax.lax.broadcasted_iota(jnp.int32, (T, E), 1)
    m1 = jnp.max(logits, axis=1, keepdims=True)
    i1 = jnp.min(jnp.where(logits == m1, ie, E), axis=1, keepdims=True)
    mask1 = ie == i1
    l2 = jnp.where(mask1, -jnp.inf, logits)
    m2 = jnp.max(l2, axis=1, keepdims=True)
    i2 = jnp.min(jnp.where(l2 == m2, ie, E), axis=1, keepdims=True)
    mask2 = ie == i2
    sel = mask1 | mask2
    scores = jnp.where(sel, jax.nn.sigmoid(logits), 0.0)
    scores_et_ref[...] = scores.T

    oh1 = mask1.astype(f32)
    oh2 = mask2.astype(f32)
    self32 = sel.astype(f32)

    scale1 = jnp.sum(oh1 * scores, axis=1, keepdims=True)
    scale2 = jnp.sum(oh2 * scores, axis=1, keepdims=True)
    h2a_ref[...] = hs * scale1
    h2b_ref[...] = hs * scale2

    cb = 512
    nb = T // cb
    rr = jax.lax.broadcasted_iota(jnp.int32, (cb, cb), 0)
    cc = jax.lax.broadcasted_iota(jnp.int32, (cb, cb), 1)
    tril_s = (cc < rr).astype(f32)
    carry = jnp.zeros((1, E), f32)
    blocks = []
    for b in range(nb):
        blk = self32[b * cb:(b + 1) * cb, :]
        blocks.append(_dot(tril_s, blk) + carry)
        carry = carry + jnp.sum(blk, axis=0, keepdims=True)
    count_c = jnp.concatenate(blocks, axis=0)
    counts_row = carry

    ng_row = jnp.floor((counts_row + (bt - 1)) * (1.0 / bt))
    er = jax.lax.broadcasted_iota(jnp.int32, (E, E), 0)
    ec = jax.lax.broadcasted_iota(jnp.int32, (E, E), 1)
    triu_s = (er < ec).astype(f32)
    gstart_row = _dot(ng_row, triu_s)
    off_row = gstart_row * bt

    slot_base = count_c + off_row
    slot1 = jnp.sum(oh1 * slot_base, axis=1, keepdims=True)
    slot2 = jnp.sum(oh2 * slot_base, axis=1, keepdims=True)
    slots_ref[...] = jnp.concatenate(
        [slot1, slot2], axis=1).astype(jnp.int32)

    ones_t = jnp.ones((T, 1), f32)
    counts_col = jax.lax.dot_general(
        self32, ones_t, (((0,), (0,)), ((), ())),
        preferred_element_type=f32)
    ng_col = jnp.floor((counts_col + (bt - 1)) * (1.0 / bt))
    tril_se = (ec < er).astype(f32)
    gstart_col = _dot(tril_se, ng_col)
    gg = jax.lax.broadcasted_iota(jnp.int32, (E, g_max), 1).astype(f32)
    m = (gstart_col <= gg).astype(f32)
    ones_e = jnp.ones((1, E), f32)
    ge_row = _dot(ones_e, m) - 1.0
    n_used = _dot(ones_e, ng_col)
    ge_ref[...] = ge_row.astype(jnp.int32)
    nu_ref[...] = n_used.astype(jnp.int32)


def _dispatch_body(h2a_ref, h2b_ref, s1_ref, s2_ref, x_ref, ibuf, rbuf,
                   sem):
    wid = lax.axis_index("s") * _NC + lax.axis_index("c")
    tpw = s1_ref.shape[0] // _NW
    base = wid * tpw
    for ch in range(tpw // _L):
        rows = pl.ds(base + ch * _L, _L)
        pltpu.sync_copy(s1_ref.at[rows], ibuf)
        pltpu.sync_copy(h2a_ref.at[rows], rbuf)
        pltpu.async_copy(rbuf, x_ref.at[ibuf], sem).wait()
        pltpu.sync_copy(s2_ref.at[rows], ibuf)
        pltpu.sync_copy(h2b_ref.at[rows], rbuf)
        pltpu.async_copy(rbuf, x_ref.at[ibuf], sem).wait()


def _gather_body(y_ref, s1_ref, s2_ref, g_ref, ibuf, rbuf, sem):
    wid = lax.axis_index("s") * _NC + lax.axis_index("c")
    T = s1_ref.shape[0]
    tpw = T // _NW
    base = wid * tpw
    for ch in range(tpw // _L):
        rows = pl.ds(base + ch * _L, _L)
        pltpu.sync_copy(s1_ref.at[rows], ibuf)
        pltpu.async_copy(y_ref.at[ibuf], rbuf, sem).wait()
        pltpu.sync_copy(rbuf, g_ref.at[rows])
        pltpu.sync_copy(s2_ref.at[rows], ibuf)
        pltpu.async_copy(y_ref.at[ibuf], rbuf, sem).wait()
        pltpu.sync_copy(rbuf, g_ref.at[pl.ds(T + base + ch * _L, _L)])


def _shared_body(hs_ref, sg_ref, su_ref, sd_ref, out_ref, *, bt):
    f = pl.program_id(0)
    tb = pl.program_id(1)
    x = hs_ref[...]
    g = _dot(x, sg_ref[...])
    u = _dot(x, su_ref[...])
    y = _dot(_silu(g) * u, sd_ref[...])
    rows = pl.ds(tb * bt, bt)

    @pl.when(f == 0)
    def _init():
        out_ref[rows, :] = y

    @pl.when(f != 0)
    def _acc():
        out_ref[rows, :] += y


def _group_body(ge_ref, nu_ref, x_ref, gw_ref, uw_ref, dn_ref, y_ref):
    g = pl.program_id(0)
    f = pl.program_id(1)

    @pl.when(g < nu_ref[0])
    def _work():
        x = x_ref[...]
        gb = _dot(x, gw_ref[0])
        ub = _dot(x, uw_ref[0])
        yp = _dot(ub * _silu(gb), dn_ref[0])

        @pl.when(f == 0)
        def _y0():
            y_ref[...] = yp

        @pl.when(f != 0)
        def _yacc():
            y_ref[...] += yp


def _sum_body(sh_ref, g1_ref, g2_ref, out_ref):
    out_ref[...] = sh_ref[...] + g1_ref[...] + g2_ref[...]


def kernel(hidden_states, router_w, gate_up_proj, down_proj, shared_gate,
           shared_up, shared_down):
    d = hidden_states.shape[-1]
    hs = hidden_states.reshape(-1, d)
    T = hs.shape[0]
    E = router_w.shape[0]
    F = down_proj.shape[1]

    BT = 512
    G = 16
    FB = 1024
    n_f = F // FB

    scores_et, h2a, h2b, slots, ge2d, nu2d = pl.pallas_call(
        functools.partial(_router_body, bt=BT, g_max=G),
        out_shape=(
            jax.ShapeDtypeStruct((E, T), jnp.float32),
            jax.ShapeDtypeStruct((T, d), jnp.float32),
            jax.ShapeDtypeStruct((T, d), jnp.float32),
            jax.ShapeDtypeStruct((T, 2), jnp.int32),
            jax.ShapeDtypeStruct((1, G), jnp.int32),
            jax.ShapeDtypeStruct((1, 1), jnp.int32),
        ),
    )(hs, router_w.T)
    ge = ge2d.reshape(G)
    nu = nu2d.reshape(1)
    slot1 = slots[:, 0]
    slot2 = slots[:, 1]

    mesh = plsc.VectorSubcoreMesh(core_axis_name="c", subcore_axis_name="s")

    x_grp = pl.kernel(
        _dispatch_body,
        out_type=jax.ShapeDtypeStruct((G * BT, d), jnp.float32),
        mesh=mesh,
        scratch_types=[
            pltpu.VMEM((_L,), jnp.int32),
            pltpu.VMEM((_L, d), jnp.float32),
            pltpu.SemaphoreType.DMA,
        ],
    )(h2a, h2b, slot1, slot2)

    shared_out = pl.pallas_call(
        functools.partial(_shared_body, bt=BT),
        grid=(F // 512, T // BT),
        in_specs=[
            pl.BlockSpec((BT, d), lambda f, tb: (tb, 0)),
            pl.BlockSpec((d, 512), lambda f, tb: (0, f)),
            pl.BlockSpec((d, 512), lambda f, tb: (0, f)),
            pl.BlockSpec((512, d), lambda f, tb: (f, 0)),
        ],
        out_specs=pl.BlockSpec((T, d), lambda f, tb: (0, 0)),
        out_shape=jax.ShapeDtypeStruct((T, d), jnp.float32),
    )(hs, shared_gate, shared_up, shared_down)

    grid_spec = pltpu.PrefetchScalarGridSpec(
        num_scalar_prefetch=2,
        grid=(G, n_f),
        in_specs=[
            pl.BlockSpec((BT, d), lambda g, f, ge, nu: (g, 0)),
            pl.BlockSpec((1, d, FB), lambda g, f, ge, nu: (ge[g], 0, f)),
            pl.BlockSpec((1, d, FB),
                         lambda g, f, ge, nu: (ge[g], 0, n_f + f)),
            pl.BlockSpec((1, FB, d), lambda g, f, ge, nu: (ge[g], f, 0)),
        ],
        out_specs=pl.BlockSpec((BT, d), lambda g, f, ge, nu: (g, 0)),
    )
    y_grp = pl.pallas_call(
        _group_body,
        grid_spec=grid_spec,
        out_shape=jax.ShapeDtypeStruct((G * BT, d), jnp.float32),
    )(ge, nu, x_grp, gate_up_proj, gate_up_proj, down_proj)

    g2t = pl.kernel(
        _gather_body,
        out_type=jax.ShapeDtypeStruct((2 * T, d), jnp.float32),
        mesh=mesh,
        scratch_types=[
            pltpu.VMEM((_L,), jnp.int32),
            pltpu.VMEM((_L, d), jnp.float32),
            pltpu.SemaphoreType.DMA,
        ],
    )(y_grp, slot1, slot2)

    out = pl.pallas_call(
        _sum_body,
        out_shape=jax.ShapeDtypeStruct((T, d), jnp.float32),
    )(shared_out, g2t[:T], g2t[T:])

    return out, scores_et
